# baseline (speedup 1.0000x reference)
import jax
import jax.numpy as jnp
from jax import lax
from jax.experimental import pallas as pl
from jax.experimental.pallas import tpu as pltpu
from jax.experimental.pallas import tpu_sc as plsc

NUM_ACTIONS = 16
D = 128
NUM_ARGS = 3
L_TOTAL = 819200

NC = 2
NS = 16
NW = NC * NS
CHUNK = 128
GATHER = 128
NBUF = 4

ROWS_PER_W = L_TOTAL // NW
N_CHUNKS = ROWS_PER_W // CHUNK


def _build_combined(table):
    def body(t_ref, out_ref):
        t = t_ref[...]
        tb = jnp.broadcast_to(t[None], (NUM_ARGS, NUM_ACTIONS, D - 1))
        g = lax.broadcasted_iota(jnp.int32, (NUM_ARGS, NUM_ACTIONS, 1), 0).astype(jnp.float32)
        comb = jnp.concatenate([tb, g], axis=2)
        out_ref[...] = comb.reshape(NUM_ACTIONS * NUM_ARGS, D)

    return pl.pallas_call(
        body,
        out_shape=jax.ShapeDtypeStruct((NUM_ACTIONS * NUM_ARGS, D), jnp.float32),
    )(table)


def _sc_lookup_body(comb_hbm, act_hbm, arg_hbm, out_hbm, tbl_sh, a_v, g_v, *bufs):
    idx = bufs[:NBUF]
    rows = bufs[NBUF:2 * NBUF]
    sem_g = bufs[2 * NBUF:3 * NBUF]
    sem_o = bufs[3 * NBUF:4 * NBUF]
    sid = lax.axis_index("s")
    wid = sid * NC + lax.axis_index("c")
    w_base = wid * ROWS_PER_W

    @pl.when(sid == 0)
    def _stage():
        pltpu.sync_copy(comb_hbm, tbl_sh)

    plsc.subcore_barrier()

    def fire_gathers(c, b):
        base = w_base + c * CHUNK
        pltpu.sync_copy(act_hbm.at[pl.ds(base, CHUNK)], a_v)
        pltpu.sync_copy(arg_hbm.at[pl.ds(base, CHUNK)], g_v)
        for i in range(CHUNK // 16):
            a = a_v[pl.ds(i * 16, 16)]
            g = g_v[pl.ds(i * 16, 16)]
            idx[b][i // 8, pl.ds((i % 8) * 16, 16)] = g * NUM_ACTIONS + a
        for j in range(CHUNK // GATHER):
            pltpu.async_copy(
                tbl_sh.at[idx[b].at[j]],
                rows[b].at[pl.ds(j * GATHER, GATHER)],
                sem_g[b],
            )

    def drain_gathers(b):
        for j in range(CHUNK // GATHER):
            pltpu.make_async_copy(
                tbl_sh.at[idx[b].at[j]],
                rows[b].at[pl.ds(j * GATHER, GATHER)],
                sem_g[b],
            ).wait()

    def drain_out(b):
        pltpu.make_async_copy(
            rows[b], out_hbm.at[pl.ds(w_base, CHUNK)], sem_o[b]).wait()

    def step(c, b, drain_prev_out, prefetch):
        nb = (b + 1) % NBUF
        if drain_prev_out:
            drain_out(nb)
        if prefetch:
            fire_gathers(c + 1, nb)
        drain_gathers(b)
        pltpu.async_copy(
            rows[b], out_hbm.at[pl.ds(w_base + c * CHUNK, CHUNK)], sem_o[b])

    fire_gathers(0, 0)
    for c0 in range(NBUF - 1):
        step(c0, c0 % NBUF, False, True)

    start = NBUF - 1
    q = (N_CHUNKS - start - 1) // NBUF

    def group(k, carry):
        c = start + k * NBUF
        for j in range(NBUF):
            step(c + j, (start + j) % NBUF, True, True)
        return carry

    lax.fori_loop(0, q, group, 0, unroll=False)
    for c0 in range(start + q * NBUF, N_CHUNKS - 1):
        step(c0, c0 % NBUF, True, True)
    step(N_CHUNKS - 1, (N_CHUNKS - 1) % NBUF, True, False)
    for i in range(NBUF - 1):
        drain_out((N_CHUNKS - 1 - i) % NBUF)


@jax.jit
def kernel(actions, arguments, table):
    comb = _build_combined(table.astype(jnp.float32))
    act = actions.astype(jnp.int32)
    arg = arguments.astype(jnp.int32)

    mesh = plsc.VectorSubcoreMesh(core_axis_name="c", subcore_axis_name="s")
    scratch = [
        pltpu.VMEM_SHARED((NUM_ACTIONS * NUM_ARGS, D), jnp.float32),
        pltpu.VMEM((CHUNK,), jnp.int32),
        pltpu.VMEM((CHUNK,), jnp.int32),
    ]
    scratch += [pltpu.VMEM((CHUNK // GATHER, GATHER), jnp.int32)
                for _ in range(NBUF)]
    scratch += [pltpu.VMEM((CHUNK, D), jnp.float32)
                for _ in range(NBUF)]
    scratch += [pltpu.SemaphoreType.DMA for _ in range(2 * NBUF)]
    lookup = pl.kernel(
        _sc_lookup_body,
        out_type=jax.ShapeDtypeStruct((L_TOTAL, D), jnp.float32),
        mesh=mesh,
        scratch_types=scratch,
    )
    return lookup(comb, act, arg)


# speedup vs baseline: 1.4126x; 1.4126x over previous
import jax
import jax.numpy as jnp
from jax import lax
from jax.experimental import pallas as pl
from jax.experimental.pallas import tpu as pltpu
from jax.experimental.pallas import tpu_sc as plsc

NUM_ACTIONS = 16
D = 128
NUM_ARGS = 3
L_TOTAL = 819200

NC = 2
NS = 16
NW = NC * NS
CHUNK = 320
GATHER = 64
NBUF = 3

ROWS_PER_W = L_TOTAL // NW
N_CHUNKS = ROWS_PER_W // CHUNK


def _build_combined(table):
    def body(t_ref, out_ref):
        t = t_ref[...]
        tb = jnp.broadcast_to(t[None], (NUM_ARGS, NUM_ACTIONS, D - 1))
        g = lax.broadcasted_iota(jnp.int32, (NUM_ARGS, NUM_ACTIONS, 1), 0).astype(jnp.float32)
        comb = jnp.concatenate([tb, g], axis=2)
        out_ref[...] = comb.reshape(NUM_ACTIONS * NUM_ARGS, D)

    return pl.pallas_call(
        body,
        out_shape=jax.ShapeDtypeStruct((NUM_ACTIONS * NUM_ARGS, D), jnp.float32),
    )(table)


def _sc_lookup_body(comb_hbm, act_hbm, arg_hbm, out_hbm, tbl_sh, a_v, g_v, *bufs):
    idx = bufs[:NBUF]
    rows = bufs[NBUF:2 * NBUF]
    sem_g = bufs[2 * NBUF:3 * NBUF]
    sem_o = bufs[3 * NBUF:4 * NBUF]
    sid = lax.axis_index("s")
    wid = sid * NC + lax.axis_index("c")
    w_base = wid * ROWS_PER_W

    @pl.when(sid == 0)
    def _stage():
        pltpu.sync_copy(comb_hbm, tbl_sh)

    plsc.subcore_barrier()

    def fire_gathers(c, b):
        base = w_base + c * CHUNK
        pltpu.sync_copy(act_hbm.at[pl.ds(base, CHUNK)], a_v)
        pltpu.sync_copy(arg_hbm.at[pl.ds(base, CHUNK)], g_v)
        vpr = GATHER // 16
        for i in range(CHUNK // 16):
            a = a_v[pl.ds(i * 16, 16)]
            g = g_v[pl.ds(i * 16, 16)]
            idx[b][i // vpr, pl.ds((i % vpr) * 16, 16)] = g * NUM_ACTIONS + a
        for j in range(CHUNK // GATHER):
            pltpu.async_copy(
                tbl_sh.at[idx[b].at[j]],
                rows[b].at[pl.ds(j * GATHER, GATHER)],
                sem_g[b],
            )

    def drain_gathers(b):
        for j in range(CHUNK // GATHER):
            pltpu.make_async_copy(
                tbl_sh.at[idx[b].at[j]],
                rows[b].at[pl.ds(j * GATHER, GATHER)],
                sem_g[b],
            ).wait()

    def drain_out(b):
        pltpu.make_async_copy(
            rows[b], out_hbm.at[pl.ds(w_base, CHUNK)], sem_o[b]).wait()

    def step(c, b, drain_prev_out, prefetch):
        nb = (b + 1) % NBUF
        if drain_prev_out:
            drain_out(nb)
        if prefetch:
            fire_gathers(c + 1, nb)
        drain_gathers(b)
        pltpu.async_copy(
            rows[b], out_hbm.at[pl.ds(w_base + c * CHUNK, CHUNK)], sem_o[b])

    fire_gathers(0, 0)
    for c0 in range(NBUF - 1):
        step(c0, c0 % NBUF, False, True)

    start = NBUF - 1
    q = (N_CHUNKS - start - 1) // NBUF

    def group(k, carry):
        c = start + k * NBUF
        for j in range(NBUF):
            step(c + j, (start + j) % NBUF, True, True)
        return carry

    lax.fori_loop(0, q, group, 0, unroll=False)
    for c0 in range(start + q * NBUF, N_CHUNKS - 1):
        step(c0, c0 % NBUF, True, True)
    step(N_CHUNKS - 1, (N_CHUNKS - 1) % NBUF, True, False)
    for i in range(NBUF - 1):
        drain_out((N_CHUNKS - 1 - i) % NBUF)


@jax.jit
def kernel(actions, arguments, table):
    comb = _build_combined(table.astype(jnp.float32))
    act = actions.astype(jnp.int32)
    arg = arguments.astype(jnp.int32)

    mesh = plsc.VectorSubcoreMesh(core_axis_name="c", subcore_axis_name="s")
    scratch = [
        pltpu.VMEM_SHARED((NUM_ACTIONS * NUM_ARGS, D), jnp.float32),
        pltpu.VMEM((CHUNK,), jnp.int32),
        pltpu.VMEM((CHUNK,), jnp.int32),
    ]
    scratch += [pltpu.VMEM((CHUNK // GATHER, GATHER), jnp.int32)
                for _ in range(NBUF)]
    scratch += [pltpu.VMEM((CHUNK, D), jnp.float32)
                for _ in range(NBUF)]
    scratch += [pltpu.SemaphoreType.DMA for _ in range(2 * NBUF)]
    lookup = pl.kernel(
        _sc_lookup_body,
        out_type=jax.ShapeDtypeStruct((L_TOTAL, D), jnp.float32),
        mesh=mesh,
        scratch_types=scratch,
    )
    return lookup(comb, act, arg)
